# baseline (speedup 1.0000x reference)
import jax
import jax.numpy as jnp
from jax.experimental import pallas as pl
from jax.experimental.pallas import tpu as pltpu

_CB = 64


def _make_fused_kernel(B, C, H, W, k):
    nc = C // _CB

    def _fused(x_ref, out_ref, acc_ref, zeros_ref, maxv_ref, rank_ref,
               rks_ref, insem, outsem, rksem):
        b = pl.program_id(0)

        def in_cp(i):
            return pltpu.make_async_copy(
                x_ref.at[b, pl.ds(i * _CB, _CB)],
                acc_ref.at[pl.ds(i * _CB, _CB)],
                insem.at[i],
            )

        def out_chunk_cp(i):
            return pltpu.make_async_copy(
                acc_ref.at[pl.ds(i * _CB, _CB)],
                out_ref.at[b, pl.ds(i * _CB, _CB)],
                outsem.at[i],
            )

        @pl.when(b == 0)
        def _():
            zeros_ref[...] = jnp.zeros((1, H, W), zeros_ref.dtype)

        for i in range(nc):
            @pl.when(b > 0)
            def _(i=i):
                out_chunk_cp(i).wait()

            in_cp(i).start()

        for i in range(nc):
            in_cp(i).wait()
            blk = acc_ref[i * _CB:(i + 1) * _CB]
            maxv_ref[0, i * _CB:(i + 1) * _CB] = jnp.max(blk, axis=(1, 2))

        v = maxv_ref[...]
        vj = v[:, None, :]
        vc = v[:, :, None]
        j = jax.lax.broadcasted_iota(jnp.int32, (1, C, C), 2)
        c = jax.lax.broadcasted_iota(jnp.int32, (1, C, C), 1)
        beats = (vj > vc) | ((vj == vc) & (j < c))
        rank_ref[...] = jnp.sum(beats.astype(jnp.int32), axis=2)

        cp = pltpu.make_async_copy(rank_ref, rks_ref, rksem)
        cp.start()
        cp.wait()

        def wb_body(ch, carry):
            sel = rks_ref[0, ch] < k
            sem = outsem.at[ch // _CB]
            dst = out_ref.at[b, pl.ds(ch, 1)]

            @pl.when(sel)
            def _():
                pltpu.make_async_copy(acc_ref.at[pl.ds(ch, 1)], dst, sem).start()

            @pl.when(jnp.logical_not(sel))
            def _():
                pltpu.make_async_copy(zeros_ref, dst, sem).start()

            return carry

        jax.lax.fori_loop(0, C, wb_body, 0)

        @pl.when(b == B - 1)
        def _():
            for i in range(nc):
                out_chunk_cp(i).wait()

    return _fused


def kernel(x):
    B, C, H, W = x.shape
    k = C // 2

    out = pl.pallas_call(
        _make_fused_kernel(B, C, H, W, k),
        grid=(B,),
        in_specs=[pl.BlockSpec(memory_space=pl.ANY)],
        out_specs=pl.BlockSpec(memory_space=pl.ANY),
        out_shape=jax.ShapeDtypeStruct((B, C, H, W), x.dtype),
        scratch_shapes=[
            pltpu.VMEM((C, H, W), x.dtype),
            pltpu.VMEM((1, H, W), x.dtype),
            pltpu.VMEM((1, C), x.dtype),
            pltpu.VMEM((1, C), jnp.int32),
            pltpu.SMEM((1, C), jnp.int32),
            pltpu.SemaphoreType.DMA((C // _CB,)),
            pltpu.SemaphoreType.DMA((C // _CB,)),
            pltpu.SemaphoreType.DMA,
        ],
    )(x)

    return out


# speedup vs baseline: 1.0022x; 1.0022x over previous
import jax
import jax.numpy as jnp
from jax.experimental import pallas as pl
from jax.experimental.pallas import tpu as pltpu

_CB = 32


def _make_fused_kernel(B, C, H, W, k):
    nc = C // _CB

    def _fused(x_ref, out_ref, acc_ref, maxv_ref, src_ref, srcs_ref,
               insem, outsem, srcsem):
        b = pl.program_id(0)

        def in_cp(i):
            return pltpu.make_async_copy(
                x_ref.at[b, pl.ds(i * _CB, _CB)],
                acc_ref.at[pl.ds(i * _CB, _CB)],
                insem.at[i],
            )

        def out_chunk_cp(i):
            return pltpu.make_async_copy(
                acc_ref.at[pl.ds(i * _CB, _CB)],
                out_ref.at[b, pl.ds(i * _CB, _CB)],
                outsem.at[i],
            )

        @pl.when(b == 0)
        def _():
            acc_ref[pl.ds(C, 1)] = jnp.zeros((1, H, W), acc_ref.dtype)

        for i in range(nc):
            @pl.when(b > 0)
            def _(i=i):
                out_chunk_cp(i).wait()

            in_cp(i).start()

        for i in range(nc):
            in_cp(i).wait()
            blk = acc_ref[i * _CB:(i + 1) * _CB]
            maxv_ref[0, i * _CB:(i + 1) * _CB] = jnp.max(blk, axis=(1, 2))

        v = maxv_ref[...]
        vj = v[:, None, :]
        vc = v[:, :, None]
        j = jax.lax.broadcasted_iota(jnp.int32, (1, C, C), 2)
        c = jax.lax.broadcasted_iota(jnp.int32, (1, C, C), 1)
        beats = (vj > vc) | ((vj == vc) & (j < c))
        rank = jnp.sum(beats.astype(jnp.int32), axis=2)
        chan = jax.lax.broadcasted_iota(jnp.int32, (1, C), 1)
        src_ref[...] = jnp.where(rank < k, chan, C)

        cp = pltpu.make_async_copy(src_ref, srcs_ref, srcsem)
        cp.start()
        cp.wait()

        def wb_body(ch, carry):
            pltpu.make_async_copy(
                acc_ref.at[pl.ds(srcs_ref[0, ch], 1)],
                out_ref.at[b, pl.ds(ch, 1)],
                outsem.at[ch // _CB],
            ).start()
            return carry

        jax.lax.fori_loop(0, C, wb_body, 0)

        @pl.when(b == B - 1)
        def _():
            for i in range(nc):
                out_chunk_cp(i).wait()

    return _fused


def kernel(x):
    B, C, H, W = x.shape
    k = C // 2

    out = pl.pallas_call(
        _make_fused_kernel(B, C, H, W, k),
        grid=(B,),
        in_specs=[pl.BlockSpec(memory_space=pl.ANY)],
        out_specs=pl.BlockSpec(memory_space=pl.ANY),
        out_shape=jax.ShapeDtypeStruct((B, C, H, W), x.dtype),
        scratch_shapes=[
            pltpu.VMEM((C + 1, H, W), x.dtype),
            pltpu.VMEM((1, C), x.dtype),
            pltpu.VMEM((1, C), jnp.int32),
            pltpu.SMEM((1, C), jnp.int32),
            pltpu.SemaphoreType.DMA((C // _CB,)),
            pltpu.SemaphoreType.DMA((C // _CB,)),
            pltpu.SemaphoreType.DMA,
        ],
    )(x)

    return out


# speedup vs baseline: 1.0173x; 1.0151x over previous
import jax
import jax.numpy as jnp
from jax.experimental import pallas as pl
from jax.experimental.pallas import tpu as pltpu

_CB = 32


def _make_fused_kernel(B, C, H, W, k):
    nc = C // _CB
    nslot = nc + 1
    zplane = nslot * _CB

    def _fused(x_ref, out_ref, acc_ref, maxv_ref, src_ref, srcs_ref,
               insem, outsem, srcsem):
        b = pl.program_id(0)

        def slotbase(bb, i):
            return jax.lax.rem(bb * nc + i, nslot) * _CB

        def in_cp(bb, i):
            return pltpu.make_async_copy(
                x_ref.at[bb, pl.ds(i * _CB, _CB)],
                acc_ref.at[pl.ds(slotbase(bb, i), _CB)],
                insem.at[i],
            )

        def out_chunk_wait(i):
            pltpu.make_async_copy(
                acc_ref.at[pl.ds(0, _CB)],
                out_ref.at[b, pl.ds(0, _CB)],
                outsem.at[i],
            ).wait()

        @pl.when(b == 0)
        def _():
            acc_ref[pl.ds(zplane, 1)] = jnp.zeros((1, H, W), acc_ref.dtype)
            in_cp(0, 0).start()

        for i in range(1, nc):
            @pl.when(b > 0)
            def _(i=i):
                out_chunk_wait(i - 1)

            in_cp(b, i).start()

        for i in range(nc):
            in_cp(b, i).wait()
            blk = acc_ref[pl.ds(slotbase(b, i), _CB)]
            maxv_ref[0, i * _CB:(i + 1) * _CB] = jnp.max(blk, axis=(1, 2))

        v = maxv_ref[...]
        vj = v[:, None, :]
        vc = v[:, :, None]
        j = jax.lax.broadcasted_iota(jnp.int32, (1, C, C), 2)
        c = jax.lax.broadcasted_iota(jnp.int32, (1, C, C), 1)
        beats = (vj > vc) | ((vj == vc) & (j < c))
        rank = jnp.sum(beats.astype(jnp.int32), axis=2)
        lane = jax.lax.broadcasted_iota(jnp.int32, (1, _CB), 1)
        for i in range(nc):
            sl = rank[:, i * _CB:(i + 1) * _CB] < k
            src_ref[0, i * _CB:(i + 1) * _CB] = jnp.where(
                sl, slotbase(b, i) + lane, zplane)[0]

        cp = pltpu.make_async_copy(src_ref, srcs_ref, srcsem)
        cp.start()
        cp.wait()

        def wb_body(ch, carry):
            pltpu.make_async_copy(
                acc_ref.at[pl.ds(srcs_ref[0, ch], 1)],
                out_ref.at[b, pl.ds(ch, 1)],
                outsem.at[ch // _CB],
            ).start()
            return carry

        jax.lax.fori_loop(0, C, wb_body, 0)

        @pl.when(b < B - 1)
        def _():
            out_chunk_wait(nc - 1)
            in_cp(b + 1, 0).start()

        @pl.when(b == B - 1)
        def _():
            for i in range(nc):
                out_chunk_wait(i)

    return _fused


def kernel(x):
    B, C, H, W = x.shape
    k = C // 2
    nslot = C // _CB + 1

    out = pl.pallas_call(
        _make_fused_kernel(B, C, H, W, k),
        grid=(B,),
        in_specs=[pl.BlockSpec(memory_space=pl.ANY)],
        out_specs=pl.BlockSpec(memory_space=pl.ANY),
        out_shape=jax.ShapeDtypeStruct((B, C, H, W), x.dtype),
        scratch_shapes=[
            pltpu.VMEM((nslot * _CB + 1, H, W), x.dtype),
            pltpu.VMEM((1, C), x.dtype),
            pltpu.VMEM((1, C), jnp.int32),
            pltpu.SMEM((1, C), jnp.int32),
            pltpu.SemaphoreType.DMA((C // _CB,)),
            pltpu.SemaphoreType.DMA((C // _CB,)),
            pltpu.SemaphoreType.DMA,
        ],
    )(x)

    return out
